# baseline (speedup 1.0000x reference)
import functools

import jax
import jax.numpy as jnp
from jax import lax
from jax.experimental import pallas as pl
from jax.experimental.pallas import tpu as pltpu
from jax.experimental.pallas import tpu_sc as plsc

_B = 32
_CHUNK = 128
_CANVAS = 8192
_D = 256
_ALIGN = 8
_WIN = _CHUNK + _ALIGN


def _sc_build_merged(value, index, cache):
    mesh = plsc.VectorSubcoreMesh(core_axis_name="c", subcore_axis_name="s")

    @functools.partial(
        pl.kernel,
        mesh=mesh,
        out_type=jax.ShapeDtypeStruct((_B, _WIN, _D), jnp.float32),
        cost_estimate=pl.CostEstimate(
            flops=4_000_000, bytes_accessed=14_000_000, transcendentals=0),
        scratch_types=[
            pltpu.VMEM((_WIN, _D), jnp.float32),
            pltpu.VMEM((_CHUNK, _D), jnp.float32),
            pltpu.VMEM((16,), jnp.int32),
            pltpu.SemaphoreType.DMA,
        ],
    )
    def merge(value_hbm, index_hbm, cache_hbm, merged_hbm,
              win, val, idx_v, sem):
        wid = lax.axis_index("s") * 2 + lax.axis_index("c")
        b = wid

        pltpu.sync_copy(index_hbm, idx_v.at[pl.ds(0, 1)])
        idx = idx_v[...][0]
        base = pl.multiple_of((idx // _ALIGN) * _ALIGN, _ALIGN)
        off = idx - base

        pltpu.async_copy(cache_hbm.at[b, pl.ds(base, _WIN), :], win, sem)
        pltpu.async_copy(value_hbm.at[b], val, sem)
        pltpu.make_async_copy(
            cache_hbm.at[b, pl.ds(base, _WIN), :], win, sem).wait()
        pltpu.make_async_copy(value_hbm.at[b], val, sem).wait()

        def splice(r, carry):
            for c in range(_D // 16):
                win[off + r, pl.ds(c * 16, 16)] = val[r, pl.ds(c * 16, 16)]
            return carry

        lax.fori_loop(0, _CHUNK, splice, 0)
        pltpu.sync_copy(win, merged_hbm.at[b])

    return merge(value, index, cache)


def _copy_kernel(in_ref, out_ref):
    out_ref[...] = in_ref[...]


def _tc_bulk_copy(cache):
    return pl.pallas_call(
        _copy_kernel,
        grid=(_B,),
        in_specs=[pl.BlockSpec((1, _CANVAS, _D), lambda b: (b, 0, 0))],
        out_specs=pl.BlockSpec((1, _CANVAS, _D), lambda b: (b, 0, 0)),
        out_shape=jax.ShapeDtypeStruct((_B, _CANVAS, _D), cache.dtype),
        cost_estimate=pl.CostEstimate(
            flops=0, bytes_accessed=536_870_912, transcendentals=0),
    )(cache)


def _splice_kernel(index_ref, merged_ref, outin_ref, out_ref, sem):
    del outin_ref
    idx = index_ref[0]
    base = pl.multiple_of((idx // _ALIGN) * _ALIGN, _ALIGN)
    cp = pltpu.make_async_copy(
        merged_ref, out_ref.at[:, pl.ds(base, _WIN), :], sem)
    cp.start()
    cp.wait()


def _tc_splice(index, merged, out):
    return pl.pallas_call(
        _splice_kernel,
        in_specs=[
            pl.BlockSpec(memory_space=pltpu.SMEM),
            pl.BlockSpec(memory_space=pltpu.VMEM),
            pl.BlockSpec(memory_space=pl.ANY),
        ],
        out_specs=pl.BlockSpec(memory_space=pl.ANY),
        out_shape=jax.ShapeDtypeStruct((_B, _CANVAS, _D), out.dtype),
        input_output_aliases={2: 0},
        scratch_shapes=[pltpu.SemaphoreType.DMA],
    )(index, merged, out)


def kernel(value, index, cache):
    merged = _sc_build_merged(value, index, cache)
    out = _tc_bulk_copy(cache)
    return _tc_splice(index, merged, out)
